# baseline (speedup 1.0000x reference)
import functools

import jax
import jax.numpy as jnp
from jax import lax
from jax.experimental import pallas as pl
from jax.experimental.pallas import tpu as pltpu
from jax.experimental.pallas import tpu_sc as plsc

ROWS = 64
COLS = 32768
NC = 2
NS = 16
NW = NC * NS
RPW = ROWS // NW
LANES = 16
BLOCKS = COLS // LANES
UNROLL = 8

_mesh = plsc.VectorSubcoreMesh(core_axis_name="c", subcore_axis_name="s")


@functools.partial(
    pl.kernel,
    mesh=_mesh,
    out_type=jax.ShapeDtypeStruct((NW, LANES), jnp.int32),
    scratch_types=[
        pltpu.VMEM((RPW, COLS), jnp.float32),
        pltpu.VMEM((LANES,), jnp.int32),
        pltpu.SemaphoreType.DMA,
        pltpu.SemaphoreType.DMA,
    ],
)
def _argmax_sc(x_hbm, out_hbm, buf, obuf, sem0, sem1):
    wid = lax.axis_index("c") * NS + lax.axis_index("s")
    base_row = wid * RPW
    cp0 = pltpu.async_copy(x_hbm.at[pl.ds(base_row, 1)], buf.at[pl.ds(0, 1)], sem0)
    cp1 = pltpu.async_copy(x_hbm.at[pl.ds(base_row + 1, 1)], buf.at[pl.ds(1, 1)], sem1)

    lane = lax.iota(jnp.int32, 16)
    res = jnp.zeros((LANES,), jnp.int32)
    for r, cp in ((0, cp0), (1, cp1)):
        cp.wait()

        def body(jo, carry, r=r):
            m, bi = carry
            base = jo * (LANES * UNROLL)
            for u in range(UNROLL):
                v = buf[r, pl.ds(base + u * LANES, LANES)]
                gt = v > m
                m = jnp.where(gt, v, m)
                bi = jnp.where(gt, base + u * LANES, bi)
            return m, bi

        m, bi = lax.fori_loop(
            0, BLOCKS // UNROLL, body,
            (jnp.full((LANES,), -jnp.inf, jnp.float32),
             jnp.zeros((LANES,), jnp.int32)),
        )
        mv, iv = m, bi + lane
        for k in (8, 4, 2, 1):
            perm = lane ^ k
            pm = mv.at[perm].get(mode="promise_in_bounds")
            pi = iv.at[perm].get(mode="promise_in_bounds")
            tk = (pm > mv) | ((pm == mv) & (pi < iv))
            mv = jnp.where(tk, pm, mv)
            iv = jnp.where(tk, pi, iv)
        res = jnp.where(lane == r, iv, res)

    obuf[...] = res
    pltpu.sync_copy(obuf, out_hbm.at[wid])


def kernel(x):
    out = _argmax_sc(x)
    return out[:, :RPW].reshape(ROWS)
